# baseline (speedup 1.0000x reference)
import math

import jax
import jax.numpy as jnp
from jax.experimental import pallas as pl
from jax.experimental.pallas import tpu as pltpu

GX, GY, Z = 512, 512, 256
SIGMA = 2.0
BR = 16384
NB = (GX * GY) // BR


def _som_body(x_ref, denom_ref, w_ref, out_ref, minval, minidx):
    pb = pl.program_id(0)

    @pl.when(pb == 0)
    def _init():
        minval[0] = jnp.float32(jnp.inf)
        minidx[0] = jnp.int32(0)

    w = w_ref[...]
    d = w - x_ref[...]
    d2 = jnp.sum(d * d, axis=1)
    m = jnp.min(d2)

    @pl.when(m < minval[0])
    def _update():
        ii = jax.lax.broadcasted_iota(jnp.int32, (BR,), 0)
        li = jnp.min(jnp.where(d2 == m, ii, jnp.int32(2**30)))
        minval[0] = m
        minidx[0] = li + pb * BR

    @pl.when(pb == NB - 1)
    def _emit():
        flat = minidx[0]
        wi = (flat // GY).astype(jnp.float32)
        wj = (flat % GY).astype(jnp.float32)
        denom = denom_ref[0]
        ar = jax.lax.broadcasted_iota(jnp.int32, (GX, 1), 0).astype(jnp.float32)
        ac = jax.lax.broadcasted_iota(jnp.int32, (1, GY), 1).astype(jnp.float32)
        er = jnp.exp(-((ar - wi) ** 2) / denom)
        ec = jnp.exp(-((ac - wj) ** 2) / denom)
        out_ref[...] = er * ec


def kernel(x, t, W, gx, gy):
    time_const = 1000.0 / math.log(SIGMA)
    decay = SIGMA * jnp.exp(-t / time_const)
    denom = (2.0 * decay * decay).astype(jnp.float32).reshape(1)

    wf = W.reshape(GX * GY, Z)
    xf = x.reshape(1, Z)

    return pl.pallas_call(
        _som_body,
        grid=(NB,),
        in_specs=[
            pl.BlockSpec((1, Z), lambda i: (0, 0)),
            pl.BlockSpec(memory_space=pltpu.SMEM),
            pl.BlockSpec((BR, Z), lambda i: (i, 0)),
        ],
        out_specs=pl.BlockSpec((GX, GY), lambda i: (0, 0)),
        out_shape=jax.ShapeDtypeStruct((GX, GY), jnp.float32),
        scratch_shapes=[
            pltpu.SMEM((1,), jnp.float32),
            pltpu.SMEM((1,), jnp.int32),
        ],
    )(xf, denom, wf)


# speedup vs baseline: 1.0246x; 1.0246x over previous
import math

import jax
import jax.numpy as jnp
from jax.experimental import pallas as pl
from jax.experimental.pallas import tpu as pltpu

GX, GY, Z = 512, 512, 256
SIGMA = 2.0
BR = 16384
NB = (GX * GY) // BR


def _som_body(x_ref, denom_ref, w_ref, out_ref, minval, minidx):
    pb = pl.program_id(0)

    @pl.when(pb == 0)
    def _init():
        minval[0] = jnp.float32(jnp.inf)
        minidx[0] = jnp.int32(0)

    w = w_ref[...]
    d = w - x_ref[...]
    m = jnp.min(jnp.sum(d * d, axis=1, keepdims=True))

    @pl.when(m < minval[0])
    def _update():
        dd = w_ref[...] - x_ref[...]
        d2 = jnp.sum(dd * dd, axis=1, keepdims=True)
        m2 = jnp.min(d2)
        ii = jax.lax.broadcasted_iota(jnp.int32, (BR, 1), 0)
        li = jnp.min(jnp.where(d2 == m2, ii, jnp.int32(2**30)))
        minval[0] = m2
        minidx[0] = li + pb * BR

    @pl.when(pb == NB - 1)
    def _emit():
        flat = minidx[0]
        wi = (flat // GY).astype(jnp.float32)
        wj = (flat % GY).astype(jnp.float32)
        denom = denom_ref[0]
        ar = jax.lax.broadcasted_iota(jnp.int32, (GX, 1), 0).astype(jnp.float32)
        ac = jax.lax.broadcasted_iota(jnp.int32, (1, GY), 1).astype(jnp.float32)
        er = jnp.exp(-((ar - wi) ** 2) / denom)
        ec = jnp.exp(-((ac - wj) ** 2) / denom)
        out_ref[...] = er * ec


def kernel(x, t, W, gx, gy):
    time_const = 1000.0 / math.log(SIGMA)
    decay = SIGMA * jnp.exp(-t / time_const)
    denom = (2.0 * decay * decay).astype(jnp.float32).reshape(1)

    wf = W.reshape(GX * GY, Z)
    xf = x.reshape(1, Z)

    return pl.pallas_call(
        _som_body,
        grid=(NB,),
        in_specs=[
            pl.BlockSpec((1, Z), lambda i: (0, 0)),
            pl.BlockSpec(memory_space=pltpu.SMEM),
            pl.BlockSpec((BR, Z), lambda i: (i, 0)),
        ],
        out_specs=pl.BlockSpec((GX, GY), lambda i: (0, 0)),
        out_shape=jax.ShapeDtypeStruct((GX, GY), jnp.float32),
        scratch_shapes=[
            pltpu.SMEM((1,), jnp.float32),
            pltpu.SMEM((1,), jnp.int32),
        ],
    )(xf, denom, wf)


# speedup vs baseline: 1.1032x; 1.0767x over previous
import math

import jax
import jax.numpy as jnp
from jax.experimental import pallas as pl
from jax.experimental.pallas import tpu as pltpu

GX, GY, Z = 512, 512, 256
SIGMA = 2.0
BR = 16384
NB = (GX * GY) // BR


def _som_body(x_ref, denom_ref, w_ref, out_ref, minval, minidx):
    pb = pl.program_id(0)

    @pl.when(pb == 0)
    def _init():
        minval[0] = jnp.float32(jnp.inf)
        minidx[0] = jnp.int32(0)

    w = w_ref[...]
    d = w - x_ref[...]
    d2 = jnp.sum(d * d, axis=1, keepdims=True)
    m = jnp.min(d2)

    @pl.when(m < minval[0])
    def _update():
        ii = jax.lax.broadcasted_iota(jnp.int32, (BR, 1), 0)
        li = jnp.min(jnp.where(d2 == m, ii, jnp.int32(2**30)))
        minval[0] = m
        minidx[0] = li + pb * BR

    @pl.when(pb == NB - 1)
    def _emit():
        flat = minidx[0]
        wi = (flat // GY).astype(jnp.float32)
        wj = (flat % GY).astype(jnp.float32)
        denom = denom_ref[0]
        ar = jax.lax.broadcasted_iota(jnp.int32, (GX, 1), 0).astype(jnp.float32)
        ac = jax.lax.broadcasted_iota(jnp.int32, (1, GY), 1).astype(jnp.float32)
        er = jnp.exp(-((ar - wi) ** 2) / denom)
        ec = jnp.exp(-((ac - wj) ** 2) / denom)
        out_ref[...] = er * ec


def kernel(x, t, W, gx, gy):
    time_const = 1000.0 / math.log(SIGMA)
    decay = SIGMA * jnp.exp(-t / time_const)
    denom = (2.0 * decay * decay).astype(jnp.float32).reshape(1)

    wf = W.reshape(GX * GY, Z)
    xf = x.reshape(1, Z)

    return pl.pallas_call(
        _som_body,
        grid=(NB,),
        in_specs=[
            pl.BlockSpec((1, Z), lambda i: (0, 0)),
            pl.BlockSpec(memory_space=pltpu.SMEM),
            pl.BlockSpec((BR, Z), lambda i: (i, 0)),
        ],
        out_specs=pl.BlockSpec((GX, GY), lambda i: (0, 0)),
        out_shape=jax.ShapeDtypeStruct((GX, GY), jnp.float32),
        scratch_shapes=[
            pltpu.SMEM((1,), jnp.float32),
            pltpu.SMEM((1,), jnp.int32),
        ],
    )(xf, denom, wf)
